# baseline (speedup 1.0000x reference)
import functools

import jax
import jax.numpy as jnp
from jax import lax
from jax.experimental import pallas as pl
from jax.experimental.pallas import tpu as pltpu
from jax.experimental.pallas import tpu_sc as plsc

_NC = 2
_NS = 16
_NW = _NC * _NS
_CH = 1024


def kernel(x, C):
    B, H = x.shape
    V, D = C.shape
    N = B * H
    ipw = N // _NW
    nch = ipw // _CH
    assert ipw * _NW == N and nch * _CH == ipw
    assert nch >= 3 and nch % 2 == 1

    xf = x.reshape(N)

    mesh = plsc.VectorSubcoreMesh(
        core_axis_name="c", subcore_axis_name="s",
        num_cores=_NC, num_subcores=_NS)

    @functools.partial(
        pl.kernel,
        out_type=jax.ShapeDtypeStruct((N, D), jnp.float32),
        mesh=mesh,
        scratch_types=[
            pltpu.VMEM((2, _CH), jnp.int32),
            pltpu.VMEM((2, _CH, D), jnp.float32),
            pltpu.SemaphoreType.DMA,
            pltpu.SemaphoreType.DMA,
            pltpu.SemaphoreType.DMA,
            pltpu.SemaphoreType.DMA,
        ],
        compiler_params=pltpu.CompilerParams(use_tc_tiling_on_sc=False),
    )
    def emb(x_hbm, C_hbm, out_hbm, idx_v, rows_v, g0, g1, o0, o1):
        wid = lax.axis_index("s") * _NC + lax.axis_index("c")
        base0 = wid * ipw
        gsem = (g0, g1)
        osem = (o0, o1)

        def load(c, s):
            pltpu.sync_copy(x_hbm.at[pl.ds(base0 + c * _CH, _CH)], idx_v.at[s])

        def gfire(c, s):
            pltpu.async_copy(C_hbm.at[idx_v.at[s]], rows_v.at[s], gsem[s])

        def gdrain(s):
            pltpu.make_async_copy(
                out_hbm.at[pl.ds(0, _CH)], rows_v.at[s], gsem[s]).wait()

        def sfire(c, s):
            pltpu.async_copy(rows_v.at[s],
                             out_hbm.at[pl.ds(base0 + c * _CH, _CH)], osem[s])

        def sdrain(s):
            pltpu.make_async_copy(
                rows_v.at[s], out_hbm.at[pl.ds(0, _CH)], osem[s]).wait()

        load(0, 0)
        gfire(0, 0)
        load(1, 1)
        gfire(1, 1)
        gdrain(0)
        sfire(0, 0)
        load(2, 0)
        sdrain(0)
        gfire(2, 0)
        gdrain(1)
        sfire(1, 1)

        @pl.loop(3, nch - 1, step=2)
        def pair(c):
            load(c, 1)
            sdrain(1)
            gfire(c, 1)
            gdrain(0)
            sfire(c - 1, 0)
            load(c + 1, 0)
            sdrain(0)
            gfire(c + 1, 0)
            gdrain(1)
            sfire(c, 1)

        gdrain(0)
        sfire(nch - 1, 0)
        sdrain(1)
        sdrain(0)

    out = emb(xf, C)
    return out.reshape(B, H, D)


# speedup vs baseline: 1.6160x; 1.6160x over previous
import functools

import jax
import jax.numpy as jnp
from jax import lax
from jax.experimental import pallas as pl
from jax.experimental.pallas import tpu as pltpu
from jax.experimental.pallas import tpu_sc as plsc

_NC = 2
_NS = 16
_NW = _NC * _NS
_R = 8


def kernel(x, C):
    B, H = x.shape
    V, D = C.shape
    rpw = B // _NW
    nch = rpw // _R
    assert rpw * _NW == B and nch * _R == rpw
    assert nch >= 6 and nch % 2 == 0

    mesh = plsc.VectorSubcoreMesh(
        core_axis_name="c", subcore_axis_name="s",
        num_cores=_NC, num_subcores=_NS)

    @functools.partial(
        pl.kernel,
        out_type=jax.ShapeDtypeStruct((B, H, D), jnp.float32),
        mesh=mesh,
        scratch_types=[
            pltpu.VMEM((rpw, H), jnp.int32),
            pltpu.VMEM((2, _R, H, D), jnp.float32),
            pltpu.SemaphoreType.DMA,
            pltpu.SemaphoreType.DMA,
            pltpu.SemaphoreType.DMA,
            pltpu.SemaphoreType.DMA,
        ],
        compiler_params=pltpu.CompilerParams(use_tc_tiling_on_sc=False),
    )
    def emb(x_hbm, C_hbm, out_hbm, xslab, rows_v, g0, g1, o0, o1):
        wid = lax.axis_index("s") * _NC + lax.axis_index("c")
        row0 = wid * rpw
        gsem = (g0, g1)
        osem = (o0, o1)

        pltpu.sync_copy(x_hbm.at[pl.ds(row0, rpw)], xslab)

        def gfire(c, s):
            for r in range(_R):
                pltpu.async_copy(C_hbm.at[xslab.at[c * _R + r]],
                                 rows_v.at[s, r], gsem[s])

        def gdrain(s):
            pltpu.make_async_copy(
                out_hbm.at[pl.ds(0, _R)], rows_v.at[s], gsem[s]).wait()

        def sfire(c, s):
            pltpu.async_copy(rows_v.at[s],
                             out_hbm.at[pl.ds(row0 + c * _R, _R)], osem[s])

        def sdrain(s):
            pltpu.make_async_copy(
                rows_v.at[s], out_hbm.at[pl.ds(0, _R)], osem[s]).wait()

        gfire(0, 0)
        gfire(1, 1)
        gdrain(0)
        sfire(0, 0)
        sdrain(0)
        gfire(2, 0)
        gdrain(1)
        sfire(1, 1)

        @pl.loop(3, nch - 2, step=2)
        def pair(c):
            sdrain(1)
            gfire(c, 1)
            gdrain(0)
            sfire(c - 1, 0)
            sdrain(0)
            gfire(c + 1, 0)
            gdrain(1)
            sfire(c, 1)

        sdrain(1)
        gfire(nch - 1, 1)
        gdrain(0)
        sfire(nch - 2, 0)
        gdrain(1)
        sfire(nch - 1, 1)
        sdrain(0)
        sdrain(1)

    return emb(x, C)


# speedup vs baseline: 1.6231x; 1.0043x over previous
import functools

import jax
import jax.numpy as jnp
from jax import lax
from jax.experimental import pallas as pl
from jax.experimental.pallas import tpu as pltpu
from jax.experimental.pallas import tpu_sc as plsc

_NC = 2
_NS = 16
_NW = _NC * _NS
_R = 16


def kernel(x, C):
    B, H = x.shape
    V, D = C.shape
    rpw = B // _NW
    nch = rpw // _R
    assert rpw * _NW == B and nch * _R == rpw
    assert nch >= 6 and nch % 2 == 0

    mesh = plsc.VectorSubcoreMesh(
        core_axis_name="c", subcore_axis_name="s",
        num_cores=_NC, num_subcores=_NS)

    @functools.partial(
        pl.kernel,
        out_type=jax.ShapeDtypeStruct((B, H, D), jnp.float32),
        mesh=mesh,
        scratch_types=[
            pltpu.VMEM((rpw, H), jnp.int32),
            pltpu.VMEM((2, _R, H, D), jnp.float32),
            pltpu.SemaphoreType.DMA,
            pltpu.SemaphoreType.DMA,
            pltpu.SemaphoreType.DMA,
            pltpu.SemaphoreType.DMA,
        ],
        compiler_params=pltpu.CompilerParams(use_tc_tiling_on_sc=False),
    )
    def emb(x_hbm, C_hbm, out_hbm, xslab, rows_v, g0, g1, o0, o1):
        wid = lax.axis_index("s") * _NC + lax.axis_index("c")
        row0 = wid * rpw
        gsem = (g0, g1)
        osem = (o0, o1)

        pltpu.sync_copy(x_hbm.at[pl.ds(row0, rpw)], xslab)

        def gfire(c, s):
            for r in range(_R):
                pltpu.async_copy(C_hbm.at[xslab.at[c * _R + r]],
                                 rows_v.at[s, r], gsem[s])

        def gdrain(s):
            pltpu.make_async_copy(
                out_hbm.at[pl.ds(0, _R)], rows_v.at[s], gsem[s]).wait()

        def sfire(c, s):
            pltpu.async_copy(rows_v.at[s],
                             out_hbm.at[pl.ds(row0 + c * _R, _R)], osem[s])

        def sdrain(s):
            pltpu.make_async_copy(
                rows_v.at[s], out_hbm.at[pl.ds(0, _R)], osem[s]).wait()

        gfire(0, 0)
        gfire(1, 1)
        gdrain(0)
        sfire(0, 0)
        sdrain(0)
        gfire(2, 0)
        gdrain(1)
        sfire(1, 1)

        @pl.loop(3, nch - 2, step=2)
        def pair(c):
            sdrain(1)
            gfire(c, 1)
            gdrain(0)
            sfire(c - 1, 0)
            sdrain(0)
            gfire(c + 1, 0)
            gdrain(1)
            sfire(c, 1)

        sdrain(1)
        gfire(nch - 1, 1)
        gdrain(0)
        sfire(nch - 2, 0)
        gdrain(1)
        sfire(nch - 1, 1)
        sdrain(0)
        sdrain(1)

    return emb(x, C)
